# baseline (speedup 1.0000x reference)
import functools
import math

import jax
import jax.numpy as jnp
from jax import lax
from jax.experimental import pallas as pl
from jax.experimental.pallas import tpu as pltpu
from jax.experimental.pallas import tpu_sc as plsc

D_MODEL = 768
EMB_DIM = 64
SCALE = math.sqrt(D_MODEL)

_NC = 2
_NS = 16
_NW = _NC * _NS

_CHUNK = 128


def _sc_gather(idx, packed, n_rows, row_w):
    rows_per_w = n_rows // _NW
    n_chunks = rows_per_w // _CHUNK

    mesh = plsc.VectorSubcoreMesh(core_axis_name="c", subcore_axis_name="s")

    @functools.partial(
        pl.kernel,
        mesh=mesh,
        out_type=jax.ShapeDtypeStruct((n_rows, row_w), jnp.float32),
        compiler_params=pltpu.CompilerParams(use_tc_tiling_on_sc=True),
        scratch_types=[
            pltpu.VMEM((rows_per_w,), jnp.int32),
            pltpu.VMEM((rows_per_w, row_w), jnp.float32),
            pltpu.SemaphoreType.DMA,
        ],
    )
    def gather_kernel(idx_hbm, packed_hbm, out_hbm, idx_v, rows_v, sem):
        wid = lax.axis_index("s") * _NC + lax.axis_index("c")
        base = wid * rows_per_w
        pltpu.sync_copy(idx_hbm.at[pl.ds(base, rows_per_w)], idx_v)
        handles = []
        for j in range(n_chunks):
            sl = pl.ds(j * _CHUNK, _CHUNK)
            handles.append(
                pltpu.async_copy(packed_hbm.at[idx_v.at[sl]], rows_v.at[sl], sem)
            )
        for h in handles:
            h.wait()
        pltpu.sync_copy(rows_v, out_hbm.at[pl.ds(base, rows_per_w)])

    return gather_kernel(idx, packed)


def _tc_project(gath, par, w2, b2, n_rows, block_rows):
    def body(g_ref, p_ref, w_ref, b_ref, out_ref):
        g = g_ref[...]
        p = p_ref[...]
        lane = lax.broadcasted_iota(jnp.int32, g.shape, 1)
        mask = jnp.where(lane < EMB_DIM, 1.0 - p, p)
        acc = jnp.dot(g * mask, w_ref[...], preferred_element_type=jnp.float32)
        out_ref[...] = (acc + b_ref[...]) * SCALE

    return pl.pallas_call(
        body,
        grid=(n_rows // block_rows,),
        in_specs=[
            pl.BlockSpec((block_rows, 2 * EMB_DIM), lambda i: (i, 0)),
            pl.BlockSpec((block_rows, 1), lambda i: (i, 0)),
            pl.BlockSpec((2 * EMB_DIM, D_MODEL), lambda i: (0, 0)),
            pl.BlockSpec((1, D_MODEL), lambda i: (0, 0)),
        ],
        out_specs=pl.BlockSpec((block_rows, D_MODEL), lambda i: (i, 0)),
        out_shape=jax.ShapeDtypeStruct((n_rows, D_MODEL), jnp.float32),
    )(gath, par, w2, b2)


def kernel(x, table, W, b):
    B, L = x.shape
    n_rows = B * L
    idx = x.T.reshape(n_rows).astype(jnp.int32)
    p_idx = idx >> 1
    par = (idx & 1).astype(jnp.float32).reshape(n_rows, 1)
    packed = jnp.concatenate([table[0::2], table[1::2]], axis=1)
    gath = _sc_gather(p_idx, packed, n_rows, 2 * EMB_DIM)
    wt = W.T
    w2 = jnp.concatenate([wt, wt], axis=0)
    b2 = b.reshape(1, D_MODEL)
    out = _tc_project(gath, par, w2, b2, n_rows, block_rows=1024)
    return out.reshape(L, B, D_MODEL).transpose(1, 0, 2)


# speedup vs baseline: 6.6841x; 6.6841x over previous
import functools
import math

import jax
import jax.numpy as jnp
from jax import lax
from jax.experimental import pallas as pl
from jax.experimental.pallas import tpu as pltpu
from jax.experimental.pallas import tpu_sc as plsc

D_MODEL = 768
EMB_DIM = 64
SCALE = math.sqrt(D_MODEL)

_NC = 2
_NS = 16
_NW = _NC * _NS

_CHUNK = 128

_BK = 512


def _tc_repack(table_t, vocab):
    def body(t_ref, out_ref):
        x = t_ref[...]
        r = lax.broadcasted_iota(jnp.int32, (EMB_DIM, 2 * EMB_DIM), 0)
        c = lax.broadcasted_iota(jnp.int32, (EMB_DIM, 2 * EMB_DIM), 1)
        ident = jnp.where(r == c, 1.0, 0.0).astype(jnp.float32)
        out_ref[...] = lax.dot_general(x, ident, (((0,), (0,)), ((), ())),
                                       preferred_element_type=jnp.float32)

    return pl.pallas_call(
        body,
        grid=(vocab // _BK,),
        in_specs=[pl.BlockSpec((EMB_DIM, _BK), lambda i: (0, i))],
        out_specs=pl.BlockSpec((_BK, 2 * EMB_DIM), lambda i: (i, 0)),
        out_shape=jax.ShapeDtypeStruct((vocab, 2 * EMB_DIM), jnp.float32),
    )(table_t)


def _sc_gather(idx, packed, n_rows, row_w):
    rows_per_w = n_rows // _NW
    n_chunks = rows_per_w // _CHUNK

    mesh = plsc.VectorSubcoreMesh(core_axis_name="c", subcore_axis_name="s")

    @functools.partial(
        pl.kernel,
        mesh=mesh,
        out_type=jax.ShapeDtypeStruct((n_rows, row_w), jnp.float32),
        compiler_params=pltpu.CompilerParams(use_tc_tiling_on_sc=True),
        scratch_types=[
            pltpu.VMEM((rows_per_w,), jnp.int32),
            pltpu.VMEM((rows_per_w, row_w), jnp.float32),
            pltpu.SemaphoreType.DMA,
        ],
    )
    def gather_kernel(idx_hbm, packed_hbm, out_hbm, idx_v, rows_v, sem):
        wid = lax.axis_index("s") * _NC + lax.axis_index("c")
        base = wid * rows_per_w
        pltpu.sync_copy(idx_hbm.at[pl.ds(base, rows_per_w)], idx_v)
        handles = []
        for j in range(n_chunks):
            sl = pl.ds(j * _CHUNK, _CHUNK)
            handles.append(
                pltpu.async_copy(packed_hbm.at[idx_v.at[sl]], rows_v.at[sl], sem)
            )
        for h in handles:
            h.wait()
        pltpu.sync_copy(rows_v, out_hbm.at[pl.ds(base, rows_per_w)])

    return gather_kernel(idx, packed)


def _tc_project(gath, wt, b2, n_rows, block_rows):
    def body(g_ref, w_ref, b_ref, out_ref):
        g = g_ref[:, :EMB_DIM]
        acc = jnp.dot(g, w_ref[...], preferred_element_type=jnp.float32)
        out_ref[...] = (acc + b_ref[...]) * SCALE

    return pl.pallas_call(
        body,
        grid=(n_rows // block_rows,),
        in_specs=[
            pl.BlockSpec((block_rows, 2 * EMB_DIM), lambda i: (i, 0)),
            pl.BlockSpec((EMB_DIM, D_MODEL), lambda i: (0, 0)),
            pl.BlockSpec((1, D_MODEL), lambda i: (0, 0)),
        ],
        out_specs=pl.BlockSpec((block_rows, D_MODEL), lambda i: (i, 0)),
        out_shape=jax.ShapeDtypeStruct((n_rows, D_MODEL), jnp.float32),
    )(gath, wt, b2)


def kernel(x, table, W, b):
    B, L = x.shape
    n_rows = B * L
    vocab = table.shape[0]
    idx = x.T.reshape(n_rows).astype(jnp.int32)
    packed = _tc_repack(table.T, vocab)
    gath = _sc_gather(idx, packed, n_rows, 2 * EMB_DIM)
    wt = W.T
    b2 = b.reshape(1, D_MODEL)
    out = _tc_project(gath, wt, b2, n_rows, block_rows=1024)
    return out.reshape(L, B, D_MODEL).transpose(1, 0, 2)


# speedup vs baseline: 16.4671x; 2.4636x over previous
import functools
import math

import jax
import jax.numpy as jnp
from jax import lax
from jax.experimental import pallas as pl
from jax.experimental.pallas import tpu as pltpu
from jax.experimental.pallas import tpu_sc as plsc

D_MODEL = 768
EMB_DIM = 64
SCALE = math.sqrt(D_MODEL)

_NC = 2
_NS = 16
_NW = _NC * _NS

_CHUNK = 128

_BK = 2048


def _tc_repack(table_t, ident, vocab):
    def body(t_ref, i_ref, out_ref):
        x = t_ref[...]
        out_ref[...] = lax.dot_general(x, i_ref[...], (((0,), (0,)), ((), ())),
                                       preferred_element_type=jnp.float32)

    return pl.pallas_call(
        body,
        grid=((vocab + _BK - 1) // _BK,),
        in_specs=[
            pl.BlockSpec((EMB_DIM, _BK), lambda i: (0, i)),
            pl.BlockSpec((EMB_DIM, 2 * EMB_DIM), lambda i: (0, 0)),
        ],
        out_specs=pl.BlockSpec((_BK, 2 * EMB_DIM), lambda i: (i, 0)),
        out_shape=jax.ShapeDtypeStruct((vocab, 2 * EMB_DIM), jnp.float32),
    )(table_t, ident)


def _sc_gather(idx, packed, n_rows, row_w):
    rows_per_w = n_rows // _NW
    n_chunks = rows_per_w // _CHUNK

    mesh = plsc.VectorSubcoreMesh(core_axis_name="c", subcore_axis_name="s")

    @functools.partial(
        pl.kernel,
        mesh=mesh,
        out_type=jax.ShapeDtypeStruct((n_rows, row_w), jnp.float32),
        compiler_params=pltpu.CompilerParams(use_tc_tiling_on_sc=True),
        scratch_types=[
            pltpu.VMEM((rows_per_w,), jnp.int32),
            pltpu.VMEM((rows_per_w, row_w), jnp.float32),
            pltpu.SemaphoreType.DMA,
        ],
    )
    def gather_kernel(idx_hbm, packed_hbm, out_hbm, idx_v, rows_v, sem):
        wid = lax.axis_index("s") * _NC + lax.axis_index("c")
        base = wid * rows_per_w
        pltpu.sync_copy(idx_hbm.at[pl.ds(base, rows_per_w)], idx_v)
        handles = []
        for j in range(n_chunks):
            sl = pl.ds(j * _CHUNK, _CHUNK)
            handles.append(
                pltpu.async_copy(packed_hbm.at[idx_v.at[sl]], rows_v.at[sl], sem)
            )
        for h in handles:
            h.wait()
        pltpu.sync_copy(rows_v, out_hbm.at[pl.ds(base, rows_per_w)])

    return gather_kernel(idx, packed)


def _tc_project(gath, wt, b2, n_rows, block_rows):
    def body(g_ref, w_ref, b_ref, out_ref):
        g = g_ref[:, :EMB_DIM]
        acc = jnp.dot(g, w_ref[...], preferred_element_type=jnp.float32)
        out_ref[...] = (acc + b_ref[...]) * SCALE

    return pl.pallas_call(
        body,
        grid=(n_rows // block_rows,),
        in_specs=[
            pl.BlockSpec((block_rows, 2 * EMB_DIM), lambda i: (i, 0)),
            pl.BlockSpec((EMB_DIM, D_MODEL), lambda i: (0, 0)),
            pl.BlockSpec((1, D_MODEL), lambda i: (0, 0)),
        ],
        out_specs=pl.BlockSpec((block_rows, D_MODEL), lambda i: (i, 0)),
        out_shape=jax.ShapeDtypeStruct((n_rows, D_MODEL), jnp.float32),
    )(gath, wt, b2)


def kernel(x, table, W, b):
    B, L = x.shape
    n_rows = B * L
    vocab = table.shape[0]
    idx = x.T.reshape(n_rows).astype(jnp.int32)
    ident = jnp.concatenate(
        [jnp.eye(EMB_DIM, dtype=jnp.float32),
         jnp.zeros((EMB_DIM, EMB_DIM), jnp.float32)], axis=1)
    packed = _tc_repack(table.T, ident, vocab)
    gath = _sc_gather(idx, packed, n_rows, 2 * EMB_DIM)
    wt = W.T
    b2 = b.reshape(1, D_MODEL)
    out = _tc_project(gath, wt, b2, n_rows, block_rows=1024)
    return out.reshape(L, B, D_MODEL).transpose(1, 0, 2)


# speedup vs baseline: 17.1150x; 1.0393x over previous
import functools
import math

import jax
import jax.numpy as jnp
from jax import lax
from jax.experimental import pallas as pl
from jax.experimental.pallas import tpu as pltpu
from jax.experimental.pallas import tpu_sc as plsc

D_MODEL = 768
EMB_DIM = 64
SCALE = math.sqrt(D_MODEL)

_NC = 2
_NS = 16
_NW = _NC * _NS

_CHUNK = 128

_BK = 2048


def _tc_repack(table_t, ident, vocab):
    def body(t_ref, i_ref, out_ref):
        x = t_ref[...]
        xt = x.T
        out_ref[...] = jnp.concatenate(
            [xt, jnp.zeros((_BK, EMB_DIM), jnp.float32)], axis=1)

    return pl.pallas_call(
        body,
        grid=((vocab + _BK - 1) // _BK,),
        in_specs=[
            pl.BlockSpec((EMB_DIM, _BK), lambda i: (0, i)),
            pl.BlockSpec((EMB_DIM, 2 * EMB_DIM), lambda i: (0, 0)),
        ],
        out_specs=pl.BlockSpec((_BK, 2 * EMB_DIM), lambda i: (i, 0)),
        out_shape=jax.ShapeDtypeStruct((vocab, 2 * EMB_DIM), jnp.float32),
    )(table_t, ident)


def _sc_gather(idx, packed, n_rows, row_w):
    rows_per_w = n_rows // _NW
    n_chunks = rows_per_w // _CHUNK

    mesh = plsc.VectorSubcoreMesh(core_axis_name="c", subcore_axis_name="s")

    @functools.partial(
        pl.kernel,
        mesh=mesh,
        out_type=jax.ShapeDtypeStruct((n_rows, row_w), jnp.float32),
        compiler_params=pltpu.CompilerParams(use_tc_tiling_on_sc=True),
        scratch_types=[
            pltpu.VMEM((rows_per_w,), jnp.int32),
            pltpu.VMEM((rows_per_w, row_w), jnp.float32),
            pltpu.SemaphoreType.DMA,
        ],
    )
    def gather_kernel(idx_hbm, packed_hbm, out_hbm, idx_v, rows_v, sem):
        wid = lax.axis_index("s") * _NC + lax.axis_index("c")
        base = wid * rows_per_w
        pltpu.sync_copy(idx_hbm.at[pl.ds(base, rows_per_w)], idx_v)
        handles = []
        for j in range(n_chunks):
            sl = pl.ds(j * _CHUNK, _CHUNK)
            handles.append(
                pltpu.async_copy(packed_hbm.at[idx_v.at[sl]], rows_v.at[sl], sem)
            )
        for h in handles:
            h.wait()
        pltpu.sync_copy(rows_v, out_hbm.at[pl.ds(base, rows_per_w)])

    return gather_kernel(idx, packed)


def _tc_project(gath, wt, b2, n_rows, block_rows):
    def body(g_ref, w_ref, b_ref, out_ref):
        g = g_ref[:, :EMB_DIM]
        acc = jnp.dot(g, w_ref[...], preferred_element_type=jnp.float32)
        out_ref[...] = (acc + b_ref[...]) * SCALE

    return pl.pallas_call(
        body,
        grid=(n_rows // block_rows,),
        in_specs=[
            pl.BlockSpec((block_rows, 2 * EMB_DIM), lambda i: (i, 0)),
            pl.BlockSpec((EMB_DIM, D_MODEL), lambda i: (0, 0)),
            pl.BlockSpec((1, D_MODEL), lambda i: (0, 0)),
        ],
        out_specs=pl.BlockSpec((block_rows, D_MODEL), lambda i: (i, 0)),
        out_shape=jax.ShapeDtypeStruct((n_rows, D_MODEL), jnp.float32),
    )(gath, wt, b2)


def kernel(x, table, W, b):
    B, L = x.shape
    n_rows = B * L
    vocab = table.shape[0]
    idx = x.T.reshape(n_rows).astype(jnp.int32)
    ident = jnp.concatenate(
        [jnp.eye(EMB_DIM, dtype=jnp.float32),
         jnp.zeros((EMB_DIM, EMB_DIM), jnp.float32)], axis=1)
    packed = _tc_repack(table.T, ident, vocab)
    gath = _sc_gather(idx, packed, n_rows, 2 * EMB_DIM)
    wt = W.T
    b2 = b.reshape(1, D_MODEL)
    out = _tc_project(gath, wt, b2, n_rows, block_rows=1024)
    return out.reshape(L, B, D_MODEL).transpose(1, 0, 2)


# speedup vs baseline: 23.1736x; 1.3540x over previous
import functools
import math

import jax
import jax.numpy as jnp
from jax import lax
from jax.experimental import pallas as pl
from jax.experimental.pallas import tpu as pltpu
from jax.experimental.pallas import tpu_sc as plsc

D_MODEL = 768
EMB_DIM = 64
SCALE = math.sqrt(D_MODEL)

_NC = 2
_NS = 16
_NW = _NC * _NS

_CHUNK = 128

_BK = 2048


def _tc_repack(table_t, vocab):
    nblk = (vocab // 2 + _BK - 1) // _BK
    split = nblk * _BK
    max_blk = (vocab + _BK - 1) // _BK - 1

    def body(a_ref, b_ref, out_ref):
        out_ref[...] = jnp.concatenate([a_ref[...].T, b_ref[...].T], axis=1)

    packed = pl.pallas_call(
        body,
        grid=(nblk,),
        in_specs=[
            pl.BlockSpec((EMB_DIM, _BK), lambda i: (0, i)),
            pl.BlockSpec((EMB_DIM, _BK),
                         lambda i: (0, jnp.minimum(i + nblk, max_blk))),
        ],
        out_specs=pl.BlockSpec((_BK, 2 * EMB_DIM), lambda i: (i, 0)),
        out_shape=jax.ShapeDtypeStruct((split, 2 * EMB_DIM), jnp.float32),
    )(table_t, table_t)
    return packed, split


def _sc_gather(idx, packed, n_rows, row_w):
    rows_per_w = n_rows // _NW
    n_chunks = rows_per_w // _CHUNK

    mesh = plsc.VectorSubcoreMesh(core_axis_name="c", subcore_axis_name="s")

    @functools.partial(
        pl.kernel,
        mesh=mesh,
        out_type=jax.ShapeDtypeStruct((n_rows, row_w), jnp.float32),
        compiler_params=pltpu.CompilerParams(use_tc_tiling_on_sc=True),
        scratch_types=[
            pltpu.VMEM((rows_per_w,), jnp.int32),
            pltpu.VMEM((rows_per_w, row_w), jnp.float32),
            pltpu.SemaphoreType.DMA,
        ],
    )
    def gather_kernel(idx_hbm, packed_hbm, out_hbm, idx_v, rows_v, sem):
        wid = lax.axis_index("s") * _NC + lax.axis_index("c")
        base = wid * rows_per_w
        pltpu.sync_copy(idx_hbm.at[pl.ds(base, rows_per_w)], idx_v)
        handles = []
        for j in range(n_chunks):
            sl = pl.ds(j * _CHUNK, _CHUNK)
            handles.append(
                pltpu.async_copy(packed_hbm.at[idx_v.at[sl]], rows_v.at[sl], sem)
            )
        for h in handles:
            h.wait()
        pltpu.sync_copy(rows_v, out_hbm.at[pl.ds(base, rows_per_w)])

    return gather_kernel(idx, packed)


def _tc_project(gath, hbit, w2, b2, n_rows, block_rows):
    def body(g_ref, h_ref, w_ref, b_ref, out_ref):
        g = g_ref[...]
        h = h_ref[...]
        lane = lax.broadcasted_iota(jnp.int32, g.shape, 1)
        keep = (lane >= EMB_DIM) == (h == 1)
        g_sel = jnp.where(keep, g, 0.0)
        acc = jnp.dot(g_sel, w_ref[...], preferred_element_type=jnp.float32)
        out_ref[...] = (acc + b_ref[...]) * SCALE

    return pl.pallas_call(
        body,
        grid=(n_rows // block_rows,),
        in_specs=[
            pl.BlockSpec((block_rows, 2 * EMB_DIM), lambda i: (i, 0)),
            pl.BlockSpec((block_rows, 1), lambda i: (i, 0)),
            pl.BlockSpec((2 * EMB_DIM, D_MODEL), lambda i: (0, 0)),
            pl.BlockSpec((1, D_MODEL), lambda i: (0, 0)),
        ],
        out_specs=pl.BlockSpec((block_rows, D_MODEL), lambda i: (i, 0)),
        out_shape=jax.ShapeDtypeStruct((n_rows, D_MODEL), jnp.float32),
    )(gath, hbit, w2, b2)


def kernel(x, table, W, b):
    B, L = x.shape
    n_rows = B * L
    vocab = table.shape[0]
    idx = x.T.reshape(n_rows).astype(jnp.int32)
    packed, split = _tc_repack(table.T, vocab)
    p_idx = jnp.where(idx >= split, idx - split, idx)
    hbit = (idx >= split).astype(jnp.int32).reshape(n_rows, 1)
    gath = _sc_gather(p_idx, packed, n_rows, 2 * EMB_DIM)
    wt = W.T
    w2 = jnp.concatenate([wt, wt], axis=0)
    b2 = b.reshape(1, D_MODEL)
    out = _tc_project(gath, hbit, w2, b2, n_rows, block_rows=1024)
    return out.reshape(L, B, D_MODEL).transpose(1, 0, 2)


# speedup vs baseline: 28.4625x; 1.2282x over previous
import functools
import math

import jax
import jax.numpy as jnp
from jax import lax
from jax.experimental import pallas as pl
from jax.experimental.pallas import tpu as pltpu
from jax.experimental.pallas import tpu_sc as plsc

D_MODEL = 768
EMB_DIM = 64
SCALE = math.sqrt(D_MODEL)

_NC = 2
_NS = 16
_NW = _NC * _NS

_CHUNK = 128

_BK = 4096


def _tc_repack(table_t, vocab):
    nblk = (vocab // 2 + _BK - 1) // _BK
    split = nblk * _BK
    max_blk = (vocab + _BK - 1) // _BK - 1

    def body(a_ref, b_ref, out_ref):
        out_ref[...] = jnp.concatenate([a_ref[...].T, b_ref[...].T], axis=1)

    packed = pl.pallas_call(
        body,
        grid=(nblk,),
        in_specs=[
            pl.BlockSpec((EMB_DIM, _BK), lambda i: (0, i)),
            pl.BlockSpec((EMB_DIM, _BK),
                         lambda i: (0, jnp.minimum(i + nblk, max_blk))),
        ],
        out_specs=pl.BlockSpec((_BK, 2 * EMB_DIM), lambda i: (i, 0)),
        out_shape=jax.ShapeDtypeStruct((split, 2 * EMB_DIM), jnp.float32),
    )(table_t, table_t)
    return packed, split


def _sc_gather(idx, packed, n_rows, row_w):
    rows_per_w = n_rows // _NW
    n_chunks = rows_per_w // _CHUNK

    mesh = plsc.VectorSubcoreMesh(core_axis_name="c", subcore_axis_name="s")

    @functools.partial(
        pl.kernel,
        mesh=mesh,
        out_type=jax.ShapeDtypeStruct((n_rows, row_w), jnp.float32),
        compiler_params=pltpu.CompilerParams(use_tc_tiling_on_sc=True),
        scratch_types=[
            pltpu.VMEM((rows_per_w,), jnp.int32),
            pltpu.VMEM((rows_per_w, row_w), jnp.float32),
            pltpu.SemaphoreType.DMA,
        ],
    )
    def gather_kernel(idx_hbm, packed_hbm, out_hbm, idx_v, rows_v, sem):
        wid = lax.axis_index("s") * _NC + lax.axis_index("c")
        base = wid * rows_per_w
        pltpu.sync_copy(idx_hbm.at[pl.ds(base, rows_per_w)], idx_v)
        handles = []
        for j in range(n_chunks):
            sl = pl.ds(j * _CHUNK, _CHUNK)
            handles.append(
                pltpu.async_copy(packed_hbm.at[idx_v.at[sl]], rows_v.at[sl], sem)
            )
        for h in handles:
            h.wait()
        pltpu.sync_copy(rows_v, out_hbm.at[pl.ds(base, rows_per_w)])

    return gather_kernel(idx, packed)


def _tc_project(gath, hbit, w2, b2, n_rows, block_rows):
    def body(g_ref, h_ref, w_ref, b_ref, out_ref):
        g = g_ref[...]
        h = h_ref[...]
        lane = lax.broadcasted_iota(jnp.int32, g.shape, 1)
        keep = (lane >= EMB_DIM) == (h == 1)
        g_sel = jnp.where(keep, g, 0.0)
        acc = jnp.dot(g_sel, w_ref[...], preferred_element_type=jnp.float32)
        out_ref[...] = (acc + b_ref[...]) * SCALE

    return pl.pallas_call(
        body,
        grid=(n_rows // block_rows,),
        in_specs=[
            pl.BlockSpec((block_rows, 2 * EMB_DIM), lambda i: (i, 0)),
            pl.BlockSpec((block_rows, 1), lambda i: (i, 0)),
            pl.BlockSpec((2 * EMB_DIM, D_MODEL), lambda i: (0, 0)),
            pl.BlockSpec((1, D_MODEL), lambda i: (0, 0)),
        ],
        out_specs=pl.BlockSpec((block_rows, D_MODEL), lambda i: (i, 0)),
        out_shape=jax.ShapeDtypeStruct((n_rows, D_MODEL), jnp.float32),
    )(gath, hbit, w2, b2)


def kernel(x, table, W, b):
    B, L = x.shape
    n_rows = B * L
    vocab = table.shape[0]
    idx = x.T.reshape(n_rows).astype(jnp.int32)
    packed, split = _tc_repack(table.T, vocab)
    p_idx = jnp.where(idx >= split, idx - split, idx)
    hbit = (idx >= split).astype(jnp.int32).reshape(n_rows, 1)
    gath = _sc_gather(p_idx, packed, n_rows, 2 * EMB_DIM)
    wt = W.T
    w2 = jnp.concatenate([wt, wt], axis=0)
    b2 = b.reshape(1, D_MODEL)
    out = _tc_project(gath, hbit, w2, b2, n_rows, block_rows=2048)
    return out.reshape(L, B, D_MODEL).transpose(1, 0, 2)


# speedup vs baseline: 31.7096x; 1.1141x over previous
import functools
import math

import jax
import jax.numpy as jnp
from jax import lax
from jax.experimental import pallas as pl
from jax.experimental.pallas import tpu as pltpu
from jax.experimental.pallas import tpu_sc as plsc

D_MODEL = 768
EMB_DIM = 64
SCALE = math.sqrt(D_MODEL)

_NC = 2
_NS = 16
_NW = _NC * _NS

_CHUNK = 128

_BK = 8192


def _tc_repack(table_t, vocab):
    nblk = (vocab // 2 + _BK - 1) // _BK
    split = nblk * _BK
    max_blk = (vocab + _BK - 1) // _BK - 1

    def body(a_ref, b_ref, out_ref):
        out_ref[...] = jnp.concatenate([a_ref[...].T, b_ref[...].T], axis=1)

    packed = pl.pallas_call(
        body,
        grid=(nblk,),
        in_specs=[
            pl.BlockSpec((EMB_DIM, _BK), lambda i: (0, i)),
            pl.BlockSpec((EMB_DIM, _BK),
                         lambda i: (0, jnp.minimum(i + nblk, max_blk))),
        ],
        out_specs=pl.BlockSpec((_BK, 2 * EMB_DIM), lambda i: (i, 0)),
        out_shape=jax.ShapeDtypeStruct((split, 2 * EMB_DIM), jnp.float32),
    )(table_t, table_t)
    return packed, split


def _sc_gather(idx, packed, n_rows, row_w):
    rows_per_w = n_rows // _NW
    n_chunks = rows_per_w // _CHUNK

    mesh = plsc.VectorSubcoreMesh(core_axis_name="c", subcore_axis_name="s")

    @functools.partial(
        pl.kernel,
        mesh=mesh,
        out_type=jax.ShapeDtypeStruct((n_rows, row_w), jnp.float32),
        compiler_params=pltpu.CompilerParams(use_tc_tiling_on_sc=True),
        scratch_types=[
            pltpu.VMEM((rows_per_w,), jnp.int32),
            pltpu.VMEM((rows_per_w, row_w), jnp.float32),
            pltpu.SemaphoreType.DMA,
        ],
    )
    def gather_kernel(idx_hbm, packed_hbm, out_hbm, idx_v, rows_v, sem):
        wid = lax.axis_index("s") * _NC + lax.axis_index("c")
        base = wid * rows_per_w
        pltpu.sync_copy(idx_hbm.at[pl.ds(base, rows_per_w)], idx_v)
        handles = []
        for j in range(n_chunks):
            sl = pl.ds(j * _CHUNK, _CHUNK)
            handles.append(
                pltpu.async_copy(packed_hbm.at[idx_v.at[sl]], rows_v.at[sl], sem)
            )
        for h in handles:
            h.wait()
        pltpu.sync_copy(rows_v, out_hbm.at[pl.ds(base, rows_per_w)])

    return gather_kernel(idx, packed)


def _tc_project(gath, hbit, w2, b2, n_rows, block_rows):
    def body(g_ref, h_ref, w_ref, b_ref, out_ref):
        g = g_ref[...]
        h = h_ref[...]
        lane = lax.broadcasted_iota(jnp.int32, g.shape, 1)
        keep = (lane >= EMB_DIM) == (h == 1)
        g_sel = jnp.where(keep, g, 0.0)
        acc = jnp.dot(g_sel, w_ref[...], preferred_element_type=jnp.float32)
        out_ref[...] = (acc + b_ref[...]) * SCALE

    return pl.pallas_call(
        body,
        grid=(n_rows // block_rows,),
        in_specs=[
            pl.BlockSpec((block_rows, 2 * EMB_DIM), lambda i: (i, 0)),
            pl.BlockSpec((block_rows, 1), lambda i: (i, 0)),
            pl.BlockSpec((2 * EMB_DIM, D_MODEL), lambda i: (0, 0)),
            pl.BlockSpec((1, D_MODEL), lambda i: (0, 0)),
        ],
        out_specs=pl.BlockSpec((block_rows, D_MODEL), lambda i: (i, 0)),
        out_shape=jax.ShapeDtypeStruct((n_rows, D_MODEL), jnp.float32),
    )(gath, hbit, w2, b2)


def kernel(x, table, W, b):
    B, L = x.shape
    n_rows = B * L
    vocab = table.shape[0]
    idx = x.T.reshape(n_rows).astype(jnp.int32)
    packed, split = _tc_repack(table.T, vocab)
    p_idx = jnp.where(idx >= split, idx - split, idx)
    hbit = (idx >= split).astype(jnp.int32).reshape(n_rows, 1)
    gath = _sc_gather(p_idx, packed, n_rows, 2 * EMB_DIM)
    wt = W.T
    w2 = jnp.concatenate([wt, wt], axis=0)
    b2 = b.reshape(1, D_MODEL)
    out = _tc_project(gath, hbit, w2, b2, n_rows, block_rows=2048)
    return out.reshape(L, B, D_MODEL).transpose(1, 0, 2)


# speedup vs baseline: 33.3871x; 1.0529x over previous
import functools
import math

import jax
import jax.numpy as jnp
from jax import lax
from jax.experimental import pallas as pl
from jax.experimental.pallas import tpu as pltpu
from jax.experimental.pallas import tpu_sc as plsc

D_MODEL = 768
EMB_DIM = 64
SCALE = math.sqrt(D_MODEL)

_NC = 2
_NS = 16
_NW = _NC * _NS

_CHUNK = 128

_BK = 16384


def _tc_repack(table_t, vocab):
    nblk = (vocab // 2 + _BK - 1) // _BK
    split = nblk * _BK
    max_blk = (vocab + _BK - 1) // _BK - 1

    def body(a_ref, b_ref, out_ref):
        out_ref[...] = jnp.concatenate([a_ref[...].T, b_ref[...].T], axis=1)

    packed = pl.pallas_call(
        body,
        grid=(nblk,),
        in_specs=[
            pl.BlockSpec((EMB_DIM, _BK), lambda i: (0, i)),
            pl.BlockSpec((EMB_DIM, _BK),
                         lambda i: (0, jnp.minimum(i + nblk, max_blk))),
        ],
        out_specs=pl.BlockSpec((_BK, 2 * EMB_DIM), lambda i: (i, 0)),
        out_shape=jax.ShapeDtypeStruct((split, 2 * EMB_DIM), jnp.float32),
    )(table_t, table_t)
    return packed, split


def _sc_gather(idx, packed, n_rows, row_w):
    rows_per_w = n_rows // _NW
    n_chunks = rows_per_w // _CHUNK

    mesh = plsc.VectorSubcoreMesh(core_axis_name="c", subcore_axis_name="s")

    @functools.partial(
        pl.kernel,
        mesh=mesh,
        out_type=jax.ShapeDtypeStruct((n_rows, row_w), jnp.float32),
        compiler_params=pltpu.CompilerParams(use_tc_tiling_on_sc=True),
        scratch_types=[
            pltpu.VMEM((rows_per_w,), jnp.int32),
            pltpu.VMEM((rows_per_w, row_w), jnp.float32),
            pltpu.SemaphoreType.DMA,
        ],
    )
    def gather_kernel(idx_hbm, packed_hbm, out_hbm, idx_v, rows_v, sem):
        wid = lax.axis_index("s") * _NC + lax.axis_index("c")
        base = wid * rows_per_w
        pltpu.sync_copy(idx_hbm.at[pl.ds(base, rows_per_w)], idx_v)
        handles = []
        for j in range(n_chunks):
            sl = pl.ds(j * _CHUNK, _CHUNK)
            handles.append(
                pltpu.async_copy(packed_hbm.at[idx_v.at[sl]], rows_v.at[sl], sem)
            )
        for h in handles:
            h.wait()
        pltpu.sync_copy(rows_v, out_hbm.at[pl.ds(base, rows_per_w)])

    return gather_kernel(idx, packed)


def _tc_project(gath, hbit, w2, b2, n_rows, block_rows):
    def body(g_ref, h_ref, w_ref, b_ref, out_ref):
        g = g_ref[...]
        h = h_ref[...]
        lane = lax.broadcasted_iota(jnp.int32, g.shape, 1)
        keep = (lane >= EMB_DIM) == (h == 1)
        g_sel = jnp.where(keep, g, 0.0)
        acc = jnp.dot(g_sel, w_ref[...], preferred_element_type=jnp.float32)
        out_ref[...] = (acc + b_ref[...]) * SCALE

    return pl.pallas_call(
        body,
        grid=(n_rows // block_rows,),
        in_specs=[
            pl.BlockSpec((block_rows, 2 * EMB_DIM), lambda i: (i, 0)),
            pl.BlockSpec((block_rows, 1), lambda i: (i, 0)),
            pl.BlockSpec((2 * EMB_DIM, D_MODEL), lambda i: (0, 0)),
            pl.BlockSpec((1, D_MODEL), lambda i: (0, 0)),
        ],
        out_specs=pl.BlockSpec((block_rows, D_MODEL), lambda i: (i, 0)),
        out_shape=jax.ShapeDtypeStruct((n_rows, D_MODEL), jnp.float32),
    )(gath, hbit, w2, b2)


def kernel(x, table, W, b):
    B, L = x.shape
    n_rows = B * L
    vocab = table.shape[0]
    idx = x.T.reshape(n_rows).astype(jnp.int32)
    packed, split = _tc_repack(table.T, vocab)
    p_idx = jnp.where(idx >= split, idx - split, idx)
    hbit = (idx >= split).astype(jnp.int32).reshape(n_rows, 1)
    gath = _sc_gather(p_idx, packed, n_rows, 2 * EMB_DIM)
    wt = W.T
    w2 = jnp.concatenate([wt, wt], axis=0)
    b2 = b.reshape(1, D_MODEL)
    out = _tc_project(gath, hbit, w2, b2, n_rows, block_rows=2048)
    return out.reshape(L, B, D_MODEL).transpose(1, 0, 2)
